# Initial kernel scaffold; baseline (speedup 1.0000x reference)
import jax
import jax.numpy as jnp
from jax.experimental import pallas as pl


def kernel(x, edge_index, batch, edge_attr, params):
    raise NotImplementedError("write your pallas kernel here")


# speedup vs baseline: 4.4557x; 4.4557x over previous
import functools

import jax
import jax.numpy as jnp
from jax import lax
from jax.experimental import pallas as pl
from jax.experimental.pallas import tpu as pltpu
from jax.experimental.pallas import tpu_sc as plsc

N0, E, D, R = 10000, 320000, 128, 4
NC, NS = 2, 16
NW = NC * NS
EK = 2000
INT_MIN = -2147483648
KLO = -1065353218
KHI = 1065353217

LEVELS = [(10000, 10240, 5000, 5120),
          (5000, 5120, 2500, 2560),
          (2500, 2560, 1250, 1280)]
BN_OF = {10240: 1024, 5120: 1024, 2560: 1280}

def _sc_mesh():
    return dict(
        mesh=plsc.VectorSubcoreMesh(core_axis_name="c", subcore_axis_name="s",
                                    num_cores=NC, num_subcores=NS),
        compiler_params=pltpu.CompilerParams(use_tc_tiling_on_sc=False,
                                             needs_layout_passes=False),
    )


def _nsegp(n):
    return ((n * R + 16 + 127) // 128) * 128


def _lrelu(v):
    return jnp.where(v > 0, v, 0.01 * v)


def _to_chunks(xw):
    P = xw.shape[0]
    return xw.reshape(P, 8, 16).transpose(1, 0, 2)


def _to_wide(xc):
    P = xc.shape[1]
    return xc.transpose(1, 0, 2).reshape(P, 8 * 16)


def _bn_call(x, gamma, beta):
    def body(x_ref, g_ref, b_ref, out_ref):
        xv = x_ref[...]
        mu = jnp.mean(xv, axis=0, keepdims=True)
        var = jnp.mean((xv - mu) ** 2, axis=0, keepdims=True)
        xn = (xv - mu) * lax.rsqrt(var + 1e-5) * g_ref[0:1, :] + b_ref[0:1, :]
        out_ref[pl.ds(0, N0), :] = xn
        out_ref[pl.ds(N0, 10240 - N0), :] = jnp.zeros((10240 - N0, D),
                                                      jnp.float32)
    return pl.pallas_call(
        body,
        out_shape=jax.ShapeDtypeStruct((10240, D), jnp.float32),
    )(x, gamma, beta)


def _seg0_call(dst2d, et2d):
    def body(d_ref, e_ref, o_ref):
        o_ref[...] = d_ref[...] * R + e_ref[...]
    return pl.pallas_call(
        body,
        out_shape=jax.ShapeDtypeStruct(dst2d.shape, jnp.int32),
    )(dst2d, et2d)


def _conv_call(xw, sums_w, cnts2, basis, comp_p, root, bias_b, pw_b,
               n_real, bn):
    P = xw.shape[0]
    grid = P // bn

    def body(xw_ref, sums_ref, cnts_ref, basis_ref, comp_ref, root_ref,
             bias_ref, pw_ref, xt_ref, score_ref, key_ref):
        i = pl.program_id(0)
        sums_b = sums_ref[...].reshape(bn, R, D)
        cb = cnts_ref[...]
        cnt = (cb[:, 0] + cb[:, 8]).reshape(bn, R)
        inv = 1.0 / jnp.maximum(cnt, 1.0)
        mean = sums_b * inv[:, :, None]
        agg = jnp.zeros((bn, D), jnp.float32)
        for r in range(R):
            w_r = comp_ref[r, 0] * basis_ref[0] + comp_ref[r, 1] * basis_ref[1]
            agg += jnp.dot(mean[:, r, :], w_r,
                           preferred_element_type=jnp.float32)
        xt = agg + jnp.dot(xw_ref[...], root_ref[...],
                           preferred_element_type=jnp.float32) + bias_ref[0:1, :]
        xt = _lrelu(xt)
        xt_ref[...] = xt
        pw = pw_ref[0:1, :]
        inv_norm = lax.rsqrt(jnp.sum(pw * pw))
        s = jnp.dot(xt, pw.reshape(D, 1),
                    preferred_element_type=jnp.float32) * inv_norm
        score = jnp.tanh(s)
        rows = i * bn + lax.broadcasted_iota(jnp.int32, (bn, 1), 0)
        valid = rows < n_real
        score = jnp.where(valid, score, 0.0)
        bits = lax.bitcast_convert_type(score, jnp.int32)
        key = jnp.where(bits >= 0, bits, bits ^ jnp.int32(0x7FFFFFFF))
        key = jnp.where(valid, key, jnp.int32(INT_MIN))
        score_ref[...] = score
        key_ref[...] = key

    return pl.pallas_call(
        body,
        grid=(grid,),
        in_specs=[
            pl.BlockSpec((bn, D), lambda i: (i, 0)),
            pl.BlockSpec((bn * R, D), lambda i: (i, 0)),
            pl.BlockSpec((bn * R, 16), lambda i: (i, 0)),
            pl.BlockSpec((2, D, D), lambda i: (0, 0, 0)),
            pl.BlockSpec((8, 8), lambda i: (0, 0)),
            pl.BlockSpec((D, D), lambda i: (0, 0)),
            pl.BlockSpec((8, D), lambda i: (0, 0)),
            pl.BlockSpec((8, D), lambda i: (0, 0)),
        ],
        out_specs=[
            pl.BlockSpec((bn, D), lambda i: (i, 0)),
            pl.BlockSpec((bn, 1), lambda i: (i, 0)),
            pl.BlockSpec((bn, 1), lambda i: (i, 0)),
        ],
        out_shape=[
            jax.ShapeDtypeStruct((P, D), jnp.float32),
            jax.ShapeDtypeStruct((P, 1), jnp.float32),
            jax.ShapeDtypeStruct((P, 1), jnp.int32),
        ],
    )(xw, sums_w, cnts2, basis, comp_p, root, bias_b, pw_b)


def _thresh_call(key, k):
    def body(key_ref, out_ref):
        s = key_ref[...]

        def step(_, lh):
            lo, hi = lh
            mid = lo + (hi - lo) // 2
            cnt = jnp.sum((s >= mid).astype(jnp.int32))
            big = cnt >= k
            return jnp.where(big, mid, lo), jnp.where(big, hi, mid)

        lo, hi = lax.fori_loop(0, 32, step,
                               (jnp.int32(KLO), jnp.int32(KHI)))
        t = lo
        c1 = jnp.sum((s > t).astype(jnp.int32))
        lanes = lax.broadcasted_iota(jnp.int32, (8, 128), 1)
        out_ref[...] = jnp.where(lanes == 0, t,
                                 jnp.where(lanes == 1, c1, 0))

    return pl.pallas_call(
        body,
        out_shape=jax.ShapeDtypeStruct((8, 128), jnp.int32),
    )(key)


def _maxmean_call(xw, k):
    P = xw.shape[0]

    def body(x_ref, out_ref):
        rows = lax.broadcasted_iota(jnp.int32, (P, 1), 0)
        valid = rows < k
        xv = x_ref[...]
        mx = jnp.max(jnp.where(valid, xv, -1e30), axis=0, keepdims=True)
        mn = jnp.sum(jnp.where(valid, xv, 0.0), axis=0, keepdims=True) / k
        out_ref[...] = jnp.broadcast_to(jnp.concatenate([mx, mn], axis=1),
                                        (8, 2 * D))

    return pl.pallas_call(
        body,
        out_shape=jax.ShapeDtypeStruct((8, 2 * D), jnp.float32),
    )(xw)


def _set2set_call(xw, w_ih, w_hh, b_ih, b_hh, k):
    P = xw.shape[0]

    def body(x_ref, wih_ref, whh_ref, bih_ref, bhh_ref, out_ref):
        rows = lax.broadcasted_iota(jnp.int32, (P, 1), 0)
        valid = rows < k
        xv = x_ref[...]
        h = jnp.zeros((1, D), jnp.float32)
        c = jnp.zeros((1, D), jnp.float32)
        q = jnp.zeros((1, 2 * D), jnp.float32)
        for _ in range(3):
            gates = (lax.dot_general(q, wih_ref[...],
                                     (((1,), (1,)), ((), ())),
                                     preferred_element_type=jnp.float32)
                     + bih_ref[0:1, :]
                     + lax.dot_general(h, whh_ref[...],
                                       (((1,), (1,)), ((), ())),
                                       preferred_element_type=jnp.float32)
                     + bhh_ref[0:1, :])
            ig = jax.nn.sigmoid(gates[:, 0:D])
            fg = jax.nn.sigmoid(gates[:, D:2 * D])
            gg = jnp.tanh(gates[:, 2 * D:3 * D])
            og = jax.nn.sigmoid(gates[:, 3 * D:4 * D])
            c = fg * c + ig * gg
            h = og * jnp.tanh(c)
            e = lax.dot_general(xv, h, (((1,), (1,)), ((), ())),
                                preferred_element_type=jnp.float32)
            e = jnp.where(valid, e, -1e30)
            m = jnp.max(e)
            a = jnp.exp(e - m)
            a = a / jnp.sum(a)
            r_vec = lax.dot_general(a, xv, (((0,), (0,)), ((), ())),
                                    preferred_element_type=jnp.float32)
            q = jnp.concatenate([h, r_vec], axis=1)
        out_ref[...] = jnp.broadcast_to(q, (8, 2 * D))

    return pl.pallas_call(
        body,
        out_shape=jax.ShapeDtypeStruct((8, 2 * D), jnp.float32),
    )(xw, w_ih, w_hh, b_ih, b_hh)


def _argmaxpool_call(xw, k):
    P = xw.shape[0]

    def body(x_ref, out_ref):
        rows = lax.broadcasted_iota(jnp.int32, (P, 1), 0)
        valid = rows < k
        xv = x_ref[...]
        v = xv[:, D - 1:D]
        vm = jnp.where(valid, v, -1e30)
        m = jnp.max(vm)
        is_m = (vm == m) & valid
        idx = jnp.min(jnp.where(is_m, rows, jnp.int32(2 ** 30)))
        onehot = (rows == idx).astype(jnp.float32)
        row = lax.dot_general(onehot, xv, (((0,), (0,)), ((), ())),
                              preferred_element_type=jnp.float32)
        out_ref[...] = jnp.broadcast_to(row, (8, D))

    return pl.pallas_call(
        body,
        out_shape=jax.ShapeDtypeStruct((8, D), jnp.float32),
    )(xw)


def _head_call(r1s, r2s, r3s, l1w, l1b, l2w, l2b, l3w, l3b):
    def body(a0, a1, a2, b0, b1, b2, c0, c1, c2,
             l1w_ref, l1b_ref, l2w_ref, l2b_ref, l3w_ref, l3b_ref, out_ref):
        r1 = (a0[0:1, :] + a1[0:1, :] + a2[0:1, :])
        r2 = (b0[0:1, :] + b1[0:1, :] + b2[0:1, :])
        r3 = (c0[0:1, :] + c1[0:1, :] + c2[0:1, :])
        h = jnp.concatenate([r1, r2, r3], axis=1)
        h = _lrelu(jnp.dot(h, l1w_ref[...],
                           preferred_element_type=jnp.float32) + l1b_ref[0:1, :])
        h = _lrelu(jnp.dot(h, l2w_ref[...],
                           preferred_element_type=jnp.float32) + l2b_ref[0:1, :64])
        h = jnp.dot(h, l3w_ref[...],
                    preferred_element_type=jnp.float32) + l3b_ref[0:1, :]
        lanes = lax.broadcasted_iota(jnp.int32, (1, 128), 1)
        hv = jnp.where(lanes < 2, h, -jnp.inf)
        m = jnp.max(hv)
        ez = jnp.where(lanes < 2, jnp.exp(hv - m), 0.0)
        lse = jnp.log(jnp.sum(ez)) + m
        out = jnp.where(lanes < 2, h - lse, 0.0)
        out_ref[...] = jnp.broadcast_to(out, (8, 128))

    return pl.pallas_call(
        body,
        out_shape=jax.ShapeDtypeStruct((8, 128), jnp.float32),
    )(*r1s, *r2s, *r3s, l1w, l1b, l2w, l2b, l3w, l3b)


def _zero_stripe(acc_sh, z_v, row0, rows_t):
    done = 0
    while done < rows_t:
        step = min(512, rows_t - done)
        pltpu.sync_copy(z_v.at[pl.ds(0, step), :],
                        acc_sh.at[pl.ds(row0 + done, step), :])
        done += step


def _copy_stripe(acc_sh, out_view, row0, rows_t):
    pltpu.sync_copy(acc_sh.at[pl.ds(row0, rows_t), :],
                    out_view.at[pl.ds(row0, rows_t), :])


def _cnt_call(segp, consts, n_cur):
    nsegp = _nsegp(n_cur)
    rows_t = nsegp // NS
    epw = E // NW
    nblk = epw // EK

    @functools.partial(
        pl.kernel,
        out_type=jax.ShapeDtypeStruct((2, nsegp, 8), jnp.float32),
        scratch_types=[
            pltpu.VMEM((EK, 8), jnp.float32),
            pltpu.VMEM((512, 8), jnp.float32),
            pltpu.VMEM((EK,), jnp.int32),
            pltpu.VMEM_SHARED((nsegp, 8), jnp.float32),
        ],
        **_sc_mesh(),
    )
    def body(segp_hbm, consts_hbm, out_hbm, ones_v, z_v, seg_v, acc_sh):
        cid = lax.axis_index("c")
        sid = lax.axis_index("s")
        pltpu.sync_copy(consts_hbm.at[pl.ds(0, EK), :], ones_v)
        pltpu.sync_copy(consts_hbm.at[pl.ds(EK, 512), :], z_v)
        _zero_stripe(acc_sh, z_v, sid * rows_t, rows_t)
        plsc.subcore_barrier()
        for b in range(nblk):
            off = cid * (E // 2) + sid * epw + b * EK
            pltpu.sync_copy(segp_hbm.at[pl.ds(off, EK)], seg_v)
            pltpu.sync_copy(ones_v, acc_sh.at[seg_v], add=True)
        plsc.subcore_barrier()
        _copy_stripe(acc_sh, out_hbm.at[cid], sid * rows_t, rows_t)

    return body(segp, consts)


def _segsum_call(xc, src, segp, zeros16, n_cur):
    nsegp = _nsegp(n_cur)
    rows_t = nsegp // NS
    ept = E // NS
    nblk = ept // EK

    @functools.partial(
        pl.kernel,
        out_type=jax.ShapeDtypeStruct((8, nsegp, 16), jnp.float32),
        scratch_types=[
            pltpu.VMEM((512, 16), jnp.float32),
            pltpu.VMEM((EK,), jnp.int32),
            pltpu.VMEM((EK,), jnp.int32),
            pltpu.VMEM((EK, 16), jnp.float32),
            pltpu.VMEM_SHARED((nsegp, 16), jnp.float32),
            pltpu.SemaphoreType.DMA,
        ],
        **_sc_mesh(),
    )
    def body(xc_hbm, src_hbm, segp_hbm, z_hbm, out_hbm,
             z_v, src_v, seg_v, rows_v, acc_sh, sem):
        cid = lax.axis_index("c")
        sid = lax.axis_index("s")
        pltpu.sync_copy(z_hbm, z_v)
        for rnd in range(4):
            q = cid + 2 * rnd
            _zero_stripe(acc_sh, z_v, sid * rows_t, rows_t)
            plsc.subcore_barrier()
            for b in range(nblk):
                off = sid * ept + b * EK
                pltpu.sync_copy(src_hbm.at[pl.ds(off, EK)], src_v)
                pltpu.sync_copy(segp_hbm.at[pl.ds(off, EK)], seg_v)
                pltpu.async_copy(xc_hbm.at[q].at[src_v], rows_v, sem).wait()
                pltpu.sync_copy(rows_v, acc_sh.at[seg_v], add=True)
            plsc.subcore_barrier()
            _copy_stripe(acc_sh, out_hbm.at[q], sid * rows_t, rows_t)
            plsc.subcore_barrier()

    return body(xc, src, segp, zeros16)


def _pool_call(key1d, score1d, tcarr, xtc, src, dst, et, segp,
               n_real, k, p_next, dummy_prev):
    P = key1d.shape[0]
    rows_per = p_next // NW
    epw = E // NW
    nblk = epw // EK
    dummy_new = k * R

    @functools.partial(
        pl.kernel,
        out_type=[
            jax.ShapeDtypeStruct((8, p_next, 16), jnp.float32),
            jax.ShapeDtypeStruct((E,), jnp.int32),
            jax.ShapeDtypeStruct((E,), jnp.int32),
            jax.ShapeDtypeStruct((E,), jnp.int32),
        ],
        scratch_types=[
            pltpu.VMEM((16,), jnp.int32),
            pltpu.VMEM((P,), jnp.int32),
            pltpu.VMEM((P,), jnp.float32),
            pltpu.VMEM((P,), jnp.int32),
            pltpu.VMEM((p_next,), jnp.int32),
            pltpu.VMEM((p_next,), jnp.float32),
            pltpu.VMEM((P,), jnp.int32),
            pltpu.VMEM((p_next // NW,), jnp.int32),
            pltpu.VMEM((p_next // NW,), jnp.float32),
            pltpu.VMEM((p_next // NW, 16), jnp.float32),
            pltpu.VMEM((EK,), jnp.int32),
            pltpu.VMEM((EK,), jnp.int32),
            pltpu.VMEM((EK,), jnp.int32),
            pltpu.VMEM((EK,), jnp.int32),
            pltpu.VMEM((EK,), jnp.int32),
            pltpu.VMEM((EK,), jnp.int32),
            pltpu.VMEM((EK,), jnp.int32),
            pltpu.VMEM_SHARED((P,), jnp.int32),
            pltpu.VMEM_SHARED((p_next,), jnp.int32),
            pltpu.VMEM_SHARED((p_next,), jnp.float32),
            pltpu.SemaphoreType.DMA,
        ],
        **_sc_mesh(),
    )
    def body(key_hbm, score_hbm, tc_hbm, xtc_hbm, src_hbm, dst_hbm, et_hbm,
             segp_hbm, xnc_out, srcn_out, dstn_out, segn_out,
             tc_v, key_v, score_v, code_v, idxsel_v, scoresel_v,
             code_v2, idx_v, sc_v, gbuf,
             s_v, d_v, e_v, sp_v, so_v, do_v, go_v,
             code_sh, idxsel_sh, scoresel_sh, sem):
        cid = lax.axis_index("c")
        sid = lax.axis_index("s")
        wid = sid * NC + cid

        @pl.when(sid == 0)
        def _phase_a():
            pltpu.sync_copy(tc_hbm, tc_v)
            pltpu.sync_copy(key_hbm, key_v)
            pltpu.sync_copy(score_hbm, score_v)
            tcv = tc_v[...]
            t = tcv[0]
            c1 = tcv[1]
            kc1 = k - c1
            zi = jnp.zeros((16,), jnp.int32)
            zf = jnp.zeros((16,), jnp.float32)

            def zinit(i, _):
                sl = pl.ds(i * 16, 16)
                idxsel_v[sl] = zi
                scoresel_v[sl] = zf
                return 0
            lax.fori_loop(0, p_next // 16, zinit, 0)

            iota16 = lax.iota(jnp.int32, 16)

            def scan(g, carry):
                carry_eq, carry_sel = carry
                sl = pl.ds(g * 16, 16)
                kv = key_v[sl]
                sv = score_v[sl]
                gt = kv > t
                eq = kv == t
                eqi = eq.astype(jnp.int32)
                exc_eq = plsc.cumsum(eqi) - eqi + carry_eq
                sel = gt | (eq & (exc_eq < kc1))
                seli = sel.astype(jnp.int32)
                pos = plsc.cumsum(seli) - seli + carry_sel
                code_v[sl] = jnp.where(sel, pos, -1)
                nidx = g * 16 + iota16
                plsc.store_scatter(idxsel_v, [pos], nidx, mask=sel)
                plsc.store_scatter(scoresel_v, [pos], sv, mask=sel)
                return (carry_eq + jnp.sum(eqi), carry_sel + jnp.sum(seli))

            lax.fori_loop(0, P // 16, scan, (jnp.int32(0), jnp.int32(0)))
            pltpu.sync_copy(code_v, code_sh)
            pltpu.sync_copy(idxsel_v, idxsel_sh)
            pltpu.sync_copy(scoresel_v, scoresel_sh)

        plsc.subcore_barrier()

        pltpu.sync_copy(code_sh, code_v2)
        row0 = wid * rows_per
        pltpu.sync_copy(idxsel_sh.at[pl.ds(row0, rows_per)], idx_v)
        pltpu.sync_copy(scoresel_sh.at[pl.ds(row0, rows_per)], sc_v)

        for c4 in range(8):
            pltpu.async_copy(xtc_hbm.at[c4].at[idx_v], gbuf, sem).wait()

            def srow(rr, _):
                s16 = plsc.load_gather(sc_v, [jnp.full((16,), rr, jnp.int32)])
                hsl = pl.ds(0, 16)
                gbuf[rr, hsl] = gbuf[rr, hsl] * s16
                return 0
            lax.fori_loop(0, rows_per, srow, 0)
            pltpu.sync_copy(gbuf, xnc_out.at[c4, pl.ds(row0, rows_per), :])

        for b in range(nblk):
            off = wid * epw + b * EK
            pltpu.sync_copy(src_hbm.at[pl.ds(off, EK)], s_v)
            pltpu.sync_copy(dst_hbm.at[pl.ds(off, EK)], d_v)
            pltpu.sync_copy(et_hbm.at[pl.ds(off, EK)], e_v)
            pltpu.sync_copy(segp_hbm.at[pl.ds(off, EK)], sp_v)

            def remap(j, _):
                sl = pl.ds(j * 16, 16)
                s16 = s_v[sl]
                d16 = d_v[sl]
                cs = plsc.load_gather(code_v2, [s16])
                cd = plsc.load_gather(code_v2, [d16])
                act = ((sp_v[sl] != dummy_prev) & (cs >= 0) & (cd >= 0))
                sn = jnp.maximum(cs, 0)
                dn = jnp.maximum(cd, 0)
                so_v[sl] = sn
                do_v[sl] = dn
                go_v[sl] = jnp.where(act, dn * R + e_v[sl], dummy_new)
                return 0
            lax.fori_loop(0, EK // 16, remap, 0)
            pltpu.sync_copy(so_v, srcn_out.at[pl.ds(off, EK)])
            pltpu.sync_copy(do_v, dstn_out.at[pl.ds(off, EK)])
            pltpu.sync_copy(go_v, segn_out.at[pl.ds(off, EK)])

    return body(key1d, score1d, tcarr, xtc, src, dst, et, segp)


def _bcast8(v):
    return jnp.broadcast_to(v.reshape(1, -1), (8, v.shape[-1]))


def kernel(x, edge_index, batch, edge_attr, params):
    src = edge_index[0]
    dst = edge_index[1]
    et = edge_attr

    x0w = _bn_call(x, _bcast8(params['bn_gamma']), _bcast8(params['bn_beta']))
    xc0 = _to_chunks(x0w)

    segp0 = _seg0_call(dst.reshape(2000, 160), et.reshape(2000, 160))
    segp0 = segp0.reshape(E)

    cnt_consts8 = jnp.concatenate(
        [jnp.ones((EK, 8), jnp.float32), jnp.zeros((512, 8), jnp.float32)], 0)
    zeros16 = jnp.zeros((512, 16), jnp.float32)

    cnt0 = _cnt_call(segp0, cnt_consts8, N0)
    sums0 = _segsum_call(xc0, src, segp0, zeros16, N0)

    def run_branch(bi):
        xw, xc, scur, dcur, segcur = x0w, xc0, src, dst, segp0
        sums, cnts = sums0, cnt0
        dummy_prev = N0 * R
        outs = []
        for li, (n_real, P, k, p_next) in enumerate(LEVELS):
            p = params['conv%d_%s' % (li + 1, bi)]
            pw = params['pool%d_%s' % (li + 1, bi)]
            if li > 0:
                cnts = _cnt_call(segcur, cnt_consts8, n_real)
                sums = _segsum_call(xc, scur, segcur, zeros16, n_real)
            nsegp = _nsegp(n_real)
            sums_w = sums.transpose(1, 0, 2).reshape(nsegp, D)
            cnts2 = cnts.transpose(1, 0, 2).reshape(nsegp, 16)
            comp_p = jnp.zeros((8, 8), jnp.float32).at[:R, :2].set(p['comp'])
            xt, score, key = _conv_call(
                xw, sums_w, cnts2, p['basis'], comp_p, p['root'],
                _bcast8(p['bias']), _bcast8(pw), n_real, BN_OF[P])
            xtc = _to_chunks(xt)
            tcarr = _thresh_call(key, k).reshape(-1)[0:16]
            xnc, scur, dcur, segcur = _pool_call(
                key.reshape(P), score.reshape(P), tcarr, xtc,
                scur, dcur, et, segcur, n_real, k, p_next, dummy_prev)
            xw = _to_wide(xnc)
            xc = xnc
            dummy_prev = k * R
            outs.append((xw, k))
        return outs

    outs1 = run_branch('1')
    outs2 = run_branch('2')
    outs3 = run_branch('3')

    r1s = [_maxmean_call(xn, k) for (xn, k) in outs1]
    s2s = params['s2s']
    r2s = [_set2set_call(xn, s2s['W_ih'], s2s['W_hh'],
                         _bcast8(s2s['b_ih']), _bcast8(s2s['b_hh']), k)
           for (xn, k) in outs2]
    r3s = [_argmaxpool_call(xn, k) for (xn, k) in outs3]

    l2b = jnp.zeros((8, 128), jnp.float32).at[:, :64].set(
        _bcast8(params['lin2_b']))
    l3w = jnp.zeros((64, 128), jnp.float32).at[:, :2].set(params['lin3_W'])
    l3b = jnp.zeros((8, 128), jnp.float32).at[:, :2].set(
        _bcast8(params['lin3_b']))
    out = _head_call(r1s, r2s, r3s,
                     params['lin1_W'], _bcast8(params['lin1_b']),
                     params['lin2_W'], l2b, l3w, l3b)
    return out[0:1, 0:2]
